# baseline (speedup 1.0000x reference)
import jax
import jax.numpy as jnp
from jax import lax
from jax.experimental import pallas as pl
from jax.experimental.pallas import tpu as pltpu

R, N = 64, 32768
LANES = 128
BR = 8
GRID = R // BR
CHUNKS = N // LANES
U = 8
T = CHUNKS // U

_INT_MAX = 2**31 - 1


def _tc_body(x_ref, o_ref):
    rvs = [jnp.full((BR, LANES), -jnp.inf, jnp.float32) for _ in range(U)]
    ris = [jnp.zeros((BR, LANES), jnp.int32) for _ in range(U)]

    def step(t, carry):
        rvs, ris = carry
        new_rvs, new_ris = [], []
        for k in range(U):
            j = t * U + k
            chunk = x_ref[:, pl.ds(j * LANES, LANES)]
            m = chunk > rvs[k]
            new_rvs.append(jnp.where(m, chunk, rvs[k]))
            new_ris.append(jnp.where(m, j, ris[k]))
        return tuple(new_rvs), tuple(new_ris)

    rvs, ris = lax.fori_loop(0, T, step, (tuple(rvs), tuple(ris)))

    lane = lax.broadcasted_iota(jnp.int32, (BR, LANES), 1)
    pairs = [(rvs[k], ris[k] * LANES + lane) for k in range(U)]
    while len(pairs) > 1:
        nxt = []
        for a in range(0, len(pairs), 2):
            (va, ia), (vb, ib) = pairs[a], pairs[a + 1]
            take_b = (vb > va) | ((vb == va) & (ib < ia))
            nxt.append((jnp.where(take_b, vb, va),
                        jnp.where(take_b, ib, ia)))
        pairs = nxt
    rv, ri = pairs[0]

    mx = jnp.max(rv, axis=1, keepdims=True)
    cand = jnp.where(rv == mx, ri, _INT_MAX)
    o_ref[...] = jnp.min(cand, axis=1)[None, None, :]


@jax.jit
def _argmax_rows(x):
    out = pl.pallas_call(
        _tc_body,
        grid=(GRID,),
        in_specs=[pl.BlockSpec((BR, N), lambda i: (i, 0))],
        out_specs=pl.BlockSpec((1, 1, BR), lambda i: (i, 0, 0)),
        out_shape=jax.ShapeDtypeStruct((GRID, 1, BR), jnp.int32),
    )(x)
    return out.reshape(R)


def kernel(x):
    return _argmax_rows(x)


# speedup vs baseline: 1.1644x; 1.1644x over previous
import jax
import jax.numpy as jnp
from jax import lax
from jax.experimental import pallas as pl
from jax.experimental.pallas import tpu as pltpu

R, N = 64, 32768
LANES = 128
BR = 8
GRID = R // BR
CHUNKS = N // LANES
U = 8
T = CHUNKS // U

_INT_MAX = 2**31 - 1


def _tc_body(x_ref, o_ref):
    rvs = [jnp.full((BR, LANES), -jnp.inf, jnp.float32) for _ in range(U)]
    ris = [jnp.zeros((BR, LANES), jnp.int32) for _ in range(U)]

    for t in range(T):
        for k in range(U):
            j = t * U + k
            chunk = x_ref[:, j * LANES:(j + 1) * LANES]
            m = chunk > rvs[k]
            rvs[k] = jnp.where(m, chunk, rvs[k])
            ris[k] = jnp.where(m, j, ris[k])

    lane = lax.broadcasted_iota(jnp.int32, (BR, LANES), 1)
    pairs = [(rvs[k], ris[k] * LANES + lane) for k in range(U)]
    while len(pairs) > 1:
        nxt = []
        for a in range(0, len(pairs), 2):
            (va, ia), (vb, ib) = pairs[a], pairs[a + 1]
            take_b = (vb > va) | ((vb == va) & (ib < ia))
            nxt.append((jnp.where(take_b, vb, va),
                        jnp.where(take_b, ib, ia)))
        pairs = nxt
    rv, ri = pairs[0]

    mx = jnp.max(rv, axis=1, keepdims=True)
    cand = jnp.where(rv == mx, ri, _INT_MAX)
    o_ref[...] = jnp.min(cand, axis=1)[None, None, :]


@jax.jit
def _argmax_rows(x):
    out = pl.pallas_call(
        _tc_body,
        grid=(GRID,),
        in_specs=[pl.BlockSpec((BR, N), lambda i: (i, 0))],
        out_specs=pl.BlockSpec((1, 1, BR), lambda i: (i, 0, 0)),
        out_shape=jax.ShapeDtypeStruct((GRID, 1, BR), jnp.int32),
    )(x)
    return out.reshape(R)


def kernel(x):
    return _argmax_rows(x)


# speedup vs baseline: 1.6100x; 1.3827x over previous
import jax
import jax.numpy as jnp
from jax import lax
from jax.experimental import pallas as pl
from jax.experimental.pallas import tpu as pltpu

R, N = 64, 32768
LANES = 128
BN = 4096
GRID = N // BN
CHUNKS = BN // LANES
U = 2

_INT_MAX = 2**31 - 1


def _tc_body(x_ref, o_ref, rv_ref, ri_ref):
    i = pl.program_id(0)

    @pl.when(i == 0)
    def _init():
        for k in range(U):
            rv_ref[k] = jnp.full((R, LANES), -jnp.inf, jnp.float32)
            ri_ref[k] = jnp.zeros((R, LANES), jnp.int32)

    rvs = [rv_ref[k] for k in range(U)]
    ris = [ri_ref[k] for k in range(U)]
    for jj in range(CHUNKS):
        k = jj % U
        chunk = x_ref[:, jj * LANES:(jj + 1) * LANES]
        m = chunk > rvs[k]
        rvs[k] = jnp.where(m, chunk, rvs[k])
        ris[k] = jnp.where(m, i * CHUNKS + jj, ris[k])
    for k in range(U):
        rv_ref[k] = rvs[k]
        ri_ref[k] = ris[k]

    @pl.when(i == GRID - 1)
    def _finish():
        lane = lax.broadcasted_iota(jnp.int32, (R, LANES), 1)
        pairs = [(rvs[k], ris[k] * LANES + lane) for k in range(U)]
        while len(pairs) > 1:
            nxt = []
            for a in range(0, len(pairs), 2):
                (va, ia), (vb, ib) = pairs[a], pairs[a + 1]
                take_b = (vb > va) | ((vb == va) & (ib < ia))
                nxt.append((jnp.where(take_b, vb, va),
                            jnp.where(take_b, ib, ia)))
            pairs = nxt
        rv, ri = pairs[0]
        mx = jnp.max(rv, axis=1, keepdims=True)
        cand = jnp.where(rv == mx, ri, _INT_MAX)
        o_ref[...] = jnp.min(cand, axis=1)[None, :]


@jax.jit
def _argmax_rows(x):
    out = pl.pallas_call(
        _tc_body,
        grid=(GRID,),
        in_specs=[pl.BlockSpec((R, BN), lambda i: (0, i))],
        out_specs=pl.BlockSpec((1, R), lambda i: (0, 0)),
        out_shape=jax.ShapeDtypeStruct((1, R), jnp.int32),
        scratch_shapes=[
            pltpu.VMEM((U, R, LANES), jnp.float32),
            pltpu.VMEM((U, R, LANES), jnp.int32),
        ],
    )(x)
    return out.reshape(R)


def kernel(x):
    return _argmax_rows(x)
